# Initial kernel scaffold; baseline (speedup 1.0000x reference)
import jax
import jax.numpy as jnp
from jax.experimental import pallas as pl


def kernel(x, edge_index, edge_attr, W1, b1, W2, b2, Wih, Whh, bih, bhh):
    raise NotImplementedError("write your pallas kernel here")


# speedup vs baseline: 3.0154x; 3.0154x over previous
import functools

import jax
import jax.numpy as jnp
from jax import lax
from jax.experimental import pallas as pl
from jax.experimental.pallas import tpu as pltpu
from jax.experimental.pallas import tpu_sc as plsc

H = 128
H2 = 256
H3 = 384

NC = 2
NS = 16
NW = NC * NS

_DN = (((1,), (1,)), ((), ()))


def _node_pre_body(x_ref, w1_ref, b1_ref, s1_ref, s2_ref):
    xb = x_ref[...]
    w1 = w1_ref[...]
    s1_ref[...] = lax.dot_general(xb, w1[:, 0:H], _DN,
                                  preferred_element_type=jnp.float32) + b1_ref[...]
    s2_ref[...] = lax.dot_general(xb, w1[:, H:H2], _DN,
                                  preferred_element_type=jnp.float32)


def _edge_mlp_body(g1_ref, g2_ref, ea_ref, w1c_ref, w2_ref, b2_ref, out_ref):
    t = g1_ref[...] + g2_ref[...] + lax.dot_general(
        ea_ref[...], w1c_ref[...], _DN, preferred_element_type=jnp.float32)
    h = jnp.maximum(t, 0.0)
    out_ref[...] = lax.dot_general(
        h, w2_ref[...], _DN, preferred_element_type=jnp.float32) + b2_ref[...]


def _gru_body(a0_ref, a1_ref, x_ref, wih_ref, whh_ref, bih_ref, bhh_ref, out_ref):
    agg = a0_ref[...] + a1_ref[...]
    xb = x_ref[...]
    gi = lax.dot_general(agg, wih_ref[...], _DN,
                         preferred_element_type=jnp.float32) + bih_ref[...]
    gh = lax.dot_general(xb, whh_ref[...], _DN,
                         preferred_element_type=jnp.float32) + bhh_ref[...]
    r = jax.nn.sigmoid(gi[:, 0:H] + gh[:, 0:H])
    z = jax.nn.sigmoid(gi[:, H:H2] + gh[:, H:H2])
    n = jnp.tanh(gi[:, H2:H3] + r * gh[:, H2:H3])
    out_ref[...] = (1.0 - z) * n + z * xb


def _make_gather(n_nodes, n_edges):
    epw = n_edges // NW
    ch = 80
    nch = epw // ch
    assert epw % ch == 0 and epw % 8 == 0
    mesh = plsc.VectorSubcoreMesh(core_axis_name="c", subcore_axis_name="s")

    @functools.partial(
        pl.kernel, mesh=mesh,
        out_type=[jax.ShapeDtypeStruct((n_edges, H), jnp.float32),
                  jax.ShapeDtypeStruct((n_edges, H), jnp.float32)],
        scratch_types=[pltpu.VMEM((ch,), jnp.int32),
                       pltpu.VMEM((ch,), jnp.int32),
                       pltpu.VMEM((ch, H), jnp.float32),
                       pltpu.VMEM((ch, H), jnp.float32),
                       pltpu.SemaphoreType.DMA,
                       pltpu.SemaphoreType.DMA],
    )
    def gather_k(s1_hbm, s2_hbm, src_hbm, dst_hbm, o1_hbm, o2_hbm,
                 idx1, idx2, buf1, buf2, sem1, sem2):
        wid = lax.axis_index("s") * NC + lax.axis_index("c")
        base0 = wid * epw

        def body(i, carry):
            base = base0 + i * ch
            pltpu.sync_copy(src_hbm.at[pl.ds(base, ch)], idx1)
            pltpu.sync_copy(dst_hbm.at[pl.ds(base, ch)], idx2)
            c1 = pltpu.async_copy(s1_hbm.at[idx1], buf1, sem1)
            c2 = pltpu.async_copy(s2_hbm.at[idx2], buf2, sem2)
            c1.wait()
            c2.wait()
            pltpu.sync_copy(buf1, o1_hbm.at[pl.ds(base, ch)])
            pltpu.sync_copy(buf2, o2_hbm.at[pl.ds(base, ch)])
            return carry

        lax.fori_loop(0, nch, body, 0)

    return gather_k


def _make_scatter(n_pad, n_edges):
    epw = n_edges // NW
    ch = 80
    nch = epw // ch
    rpt = n_pad // NS
    assert epw % ch == 0 and n_pad % NS == 0 and rpt % 8 == 0
    mesh = plsc.VectorSubcoreMesh(core_axis_name="c", subcore_axis_name="s")

    @functools.partial(
        pl.kernel, mesh=mesh,
        out_type=[jax.ShapeDtypeStruct((n_pad, H), jnp.float32),
                  jax.ShapeDtypeStruct((n_pad, H), jnp.float32)],
        scratch_types=[pltpu.VMEM((ch,), jnp.int32),
                       pltpu.VMEM((ch, H), jnp.float32),
                       pltpu.VMEM_SHARED((n_pad, H), jnp.float32)],
    )
    def scatter_k(msg_hbm, dst_hbm, zeros_hbm, agg0_hbm, agg1_hbm,
                  idxb, mbuf, acc_spmem):
        cid = lax.axis_index("c")
        sid = lax.axis_index("s")
        wid = sid * NC + cid
        base0 = wid * epw

        pltpu.sync_copy(zeros_hbm, acc_spmem.at[pl.ds(sid * rpt, rpt)])
        plsc.subcore_barrier()

        def body(i, carry):
            base = base0 + i * ch
            pltpu.sync_copy(dst_hbm.at[pl.ds(base, ch)], idxb)
            pltpu.sync_copy(msg_hbm.at[pl.ds(base, ch)], mbuf)
            pltpu.sync_copy(mbuf, acc_spmem.at[idxb], add=True)
            return carry

        lax.fori_loop(0, nch, body, 0)
        plsc.subcore_barrier()

        @pl.when(cid == 0)
        def _():
            pltpu.sync_copy(acc_spmem.at[pl.ds(sid * rpt, rpt)],
                            agg0_hbm.at[pl.ds(sid * rpt, rpt)])

        @pl.when(cid == 1)
        def _():
            pltpu.sync_copy(acc_spmem.at[pl.ds(sid * rpt, rpt)],
                            agg1_hbm.at[pl.ds(sid * rpt, rpt)])

    return scatter_k


def kernel(x, edge_index, edge_attr, W1, b1, W2, b2, Wih, Whh, bih, bhh):
    n_nodes, h = x.shape
    n_edges = edge_attr.shape[0]
    assert h == H

    src = edge_index[0].astype(jnp.int32)
    dst = edge_index[1].astype(jnp.int32)

    bn = 2000
    be = 2000
    assert n_nodes % bn == 0 and n_edges % be == 0

    f32 = jnp.float32

    s1t, s2t = pl.pallas_call(
        _node_pre_body,
        grid=(n_nodes // bn,),
        in_specs=[pl.BlockSpec((bn, H), lambda i: (i, 0)),
                  pl.BlockSpec((H, H3), lambda i: (0, 0)),
                  pl.BlockSpec((1, H), lambda i: (0, 0))],
        out_specs=[pl.BlockSpec((bn, H), lambda i: (i, 0)),
                   pl.BlockSpec((bn, H), lambda i: (i, 0))],
        out_shape=[jax.ShapeDtypeStruct((n_nodes, H), f32),
                   jax.ShapeDtypeStruct((n_nodes, H), f32)],
    )(x, W1, b1.reshape(1, H))

    g1, g2 = _make_gather(n_nodes, n_edges)(s1t, s2t, src, dst)

    msg = pl.pallas_call(
        _edge_mlp_body,
        grid=(n_edges // be,),
        in_specs=[pl.BlockSpec((be, H), lambda i: (i, 0)),
                  pl.BlockSpec((be, H), lambda i: (i, 0)),
                  pl.BlockSpec((be, H), lambda i: (i, 0)),
                  pl.BlockSpec((H, H), lambda i: (0, 0)),
                  pl.BlockSpec((H, H), lambda i: (0, 0)),
                  pl.BlockSpec((1, H), lambda i: (0, 0))],
        out_specs=pl.BlockSpec((be, H), lambda i: (i, 0)),
        out_shape=jax.ShapeDtypeStruct((n_edges, H), f32),
    )(g1, g2, edge_attr, W1[:, H2:H3], W2, b2.reshape(1, H))

    n_pad = ((n_nodes + NS * 8 - 1) // (NS * 8)) * (NS * 8)
    zeros = jnp.zeros((n_pad // NS, H), f32)
    agg0, agg1 = _make_scatter(n_pad, n_edges)(msg, dst, zeros)

    out = pl.pallas_call(
        _gru_body,
        grid=(n_nodes // bn,),
        in_specs=[pl.BlockSpec((bn, H), lambda i: (i, 0)),
                  pl.BlockSpec((bn, H), lambda i: (i, 0)),
                  pl.BlockSpec((bn, H), lambda i: (i, 0)),
                  pl.BlockSpec((H3, H), lambda i: (0, 0)),
                  pl.BlockSpec((H3, H), lambda i: (0, 0)),
                  pl.BlockSpec((1, H3), lambda i: (0, 0)),
                  pl.BlockSpec((1, H3), lambda i: (0, 0))],
        out_specs=pl.BlockSpec((bn, H), lambda i: (i, 0)),
        out_shape=jax.ShapeDtypeStruct((n_nodes, H), f32),
    )(agg0, agg1, x, Wih, Whh, bih.reshape(1, H3), bhh.reshape(1, H3))

    return out


# speedup vs baseline: 3.6937x; 1.2250x over previous
import functools

import jax
import jax.numpy as jnp
from jax import lax
from jax.experimental import pallas as pl
from jax.experimental.pallas import tpu as pltpu
from jax.experimental.pallas import tpu_sc as plsc

H = 128
H2 = 256
H3 = 384

NC = 2
NS = 16
NW = NC * NS

_DN = (((1,), (1,)), ((), ()))


def _node_pre_body(x_ref, w1_ref, b1_ref, s1_ref, s2_ref):
    xb = x_ref[...]
    w1 = w1_ref[...]
    s1_ref[...] = lax.dot_general(xb, w1[:, 0:H], _DN,
                                  preferred_element_type=jnp.float32) + b1_ref[...]
    s2_ref[...] = lax.dot_general(xb, w1[:, H:H2], _DN,
                                  preferred_element_type=jnp.float32)


def _edge_mlp_body(g1_ref, g2_ref, ea_ref, w1c_ref, w2_ref, b2_ref, out_ref):
    t = g1_ref[...] + g2_ref[...] + lax.dot_general(
        ea_ref[...], w1c_ref[...], _DN, preferred_element_type=jnp.float32)
    h = jnp.maximum(t, 0.0)
    out_ref[...] = lax.dot_general(
        h, w2_ref[...], _DN, preferred_element_type=jnp.float32) + b2_ref[...]


def _gru_body(a0_ref, a1_ref, x_ref, wih_ref, whh_ref, bih_ref, bhh_ref, out_ref):
    agg = a0_ref[...] + a1_ref[...]
    xb = x_ref[...]
    gi = lax.dot_general(agg, wih_ref[...], _DN,
                         preferred_element_type=jnp.float32) + bih_ref[...]
    gh = lax.dot_general(xb, whh_ref[...], _DN,
                         preferred_element_type=jnp.float32) + bhh_ref[...]
    r = jax.nn.sigmoid(gi[:, 0:H] + gh[:, 0:H])
    z = jax.nn.sigmoid(gi[:, H:H2] + gh[:, H:H2])
    n = jnp.tanh(gi[:, H2:H3] + r * gh[:, H2:H3])
    out_ref[...] = (1.0 - z) * n + z * xb


def _make_gather(n_nodes, n_edges):
    epw = n_edges // NW
    ch = 80
    nbuf = 5
    gsz = ch * nbuf
    ngr = epw // gsz
    assert epw % gsz == 0
    mesh = plsc.VectorSubcoreMesh(core_axis_name="c", subcore_axis_name="s")

    @functools.partial(
        pl.kernel, mesh=mesh,
        out_type=[jax.ShapeDtypeStruct((n_edges, H), jnp.float32),
                  jax.ShapeDtypeStruct((n_edges, H), jnp.float32)],
        scratch_types=[pltpu.VMEM((gsz,), jnp.int32),
                       pltpu.VMEM((gsz,), jnp.int32),
                       pltpu.VMEM((nbuf, ch, H), jnp.float32),
                       pltpu.VMEM((nbuf, ch, H), jnp.float32),
                       pltpu.SemaphoreType.DMA,
                       pltpu.SemaphoreType.DMA],
    )
    def gather_k(s1_hbm, s2_hbm, src_hbm, dst_hbm, o1_hbm, o2_hbm,
                 idx1, idx2, buf1, buf2, gsem, wsem):
        wid = lax.axis_index("s") * NC + lax.axis_index("c")
        base0 = wid * epw

        def body(g, carry):
            base = base0 + g * gsz
            pltpu.sync_copy(src_hbm.at[pl.ds(base, gsz)], idx1)
            pltpu.sync_copy(dst_hbm.at[pl.ds(base, gsz)], idx2)
            gets = []
            for b in range(nbuf):
                gets.append(pltpu.async_copy(
                    s1_hbm.at[idx1.at[pl.ds(b * ch, ch)]], buf1.at[b], gsem))
                gets.append(pltpu.async_copy(
                    s2_hbm.at[idx2.at[pl.ds(b * ch, ch)]], buf2.at[b], gsem))
            puts = []
            for b in range(nbuf):
                gets[2 * b].wait()
                gets[2 * b + 1].wait()
                puts.append(pltpu.async_copy(
                    buf1.at[b], o1_hbm.at[pl.ds(base + b * ch, ch)], wsem))
                puts.append(pltpu.async_copy(
                    buf2.at[b], o2_hbm.at[pl.ds(base + b * ch, ch)], wsem))
            for p in puts:
                p.wait()
            return carry

        lax.fori_loop(0, ngr, body, 0)

    return gather_k


def _make_scatter(n_pad, n_edges):
    epw = n_edges // NW
    ch = 40
    nbuf = 5
    gsz = ch * nbuf
    ngr = epw // gsz
    rpt = n_pad // NS
    assert epw % gsz == 0 and n_pad % NS == 0 and rpt % 8 == 0
    mesh = plsc.VectorSubcoreMesh(core_axis_name="c", subcore_axis_name="s")

    @functools.partial(
        pl.kernel, mesh=mesh,
        out_type=[jax.ShapeDtypeStruct((n_pad, H), jnp.float32),
                  jax.ShapeDtypeStruct((n_pad, H), jnp.float32)],
        scratch_types=[pltpu.VMEM((nbuf, ch), jnp.int32),
                       pltpu.VMEM((gsz, H), jnp.float32),
                       pltpu.VMEM_SHARED((n_pad, H), jnp.float32),
                       pltpu.SemaphoreType.DMA],
    )
    def scatter_k(msg_hbm, dst_hbm, zeros_hbm, agg0_hbm, agg1_hbm,
                  idxb, mbuf, acc_spmem, ssem):
        cid = lax.axis_index("c")
        sid = lax.axis_index("s")
        wid = sid * NC + cid
        base0 = wid * epw

        pltpu.sync_copy(zeros_hbm, acc_spmem.at[pl.ds(sid * rpt, rpt)])
        plsc.subcore_barrier()

        def body(g, carry):
            base = base0 + g * gsz
            for b in range(nbuf):
                pltpu.sync_copy(dst_hbm.at[pl.ds(base + b * ch, ch)],
                                idxb.at[b])
            pltpu.sync_copy(msg_hbm.at[pl.ds(base, gsz)], mbuf)
            adds = []
            for b in range(nbuf):
                adds.append(pltpu.async_copy(
                    mbuf.at[pl.ds(b * ch, ch)], acc_spmem.at[idxb.at[b]],
                    ssem, add=True))
            for a in adds:
                a.wait()
            return carry

        lax.fori_loop(0, ngr, body, 0)
        plsc.subcore_barrier()

        @pl.when(cid == 0)
        def _():
            pltpu.sync_copy(acc_spmem.at[pl.ds(sid * rpt, rpt)],
                            agg0_hbm.at[pl.ds(sid * rpt, rpt)])

        @pl.when(cid == 1)
        def _():
            pltpu.sync_copy(acc_spmem.at[pl.ds(sid * rpt, rpt)],
                            agg1_hbm.at[pl.ds(sid * rpt, rpt)])

    return scatter_k


def kernel(x, edge_index, edge_attr, W1, b1, W2, b2, Wih, Whh, bih, bhh):
    n_nodes, h = x.shape
    n_edges = edge_attr.shape[0]
    assert h == H

    src = edge_index[0].astype(jnp.int32)
    dst = edge_index[1].astype(jnp.int32)

    bn = 2000
    be = 2000
    assert n_nodes % bn == 0 and n_edges % be == 0

    f32 = jnp.float32

    s1t, s2t = pl.pallas_call(
        _node_pre_body,
        grid=(n_nodes // bn,),
        in_specs=[pl.BlockSpec((bn, H), lambda i: (i, 0)),
                  pl.BlockSpec((H, H3), lambda i: (0, 0)),
                  pl.BlockSpec((1, H), lambda i: (0, 0))],
        out_specs=[pl.BlockSpec((bn, H), lambda i: (i, 0)),
                   pl.BlockSpec((bn, H), lambda i: (i, 0))],
        out_shape=[jax.ShapeDtypeStruct((n_nodes, H), f32),
                   jax.ShapeDtypeStruct((n_nodes, H), f32)],
    )(x, W1, b1.reshape(1, H))

    g1, g2 = _make_gather(n_nodes, n_edges)(s1t, s2t, src, dst)

    msg = pl.pallas_call(
        _edge_mlp_body,
        grid=(n_edges // be,),
        in_specs=[pl.BlockSpec((be, H), lambda i: (i, 0)),
                  pl.BlockSpec((be, H), lambda i: (i, 0)),
                  pl.BlockSpec((be, H), lambda i: (i, 0)),
                  pl.BlockSpec((H, H), lambda i: (0, 0)),
                  pl.BlockSpec((H, H), lambda i: (0, 0)),
                  pl.BlockSpec((1, H), lambda i: (0, 0))],
        out_specs=pl.BlockSpec((be, H), lambda i: (i, 0)),
        out_shape=jax.ShapeDtypeStruct((n_edges, H), f32),
    )(g1, g2, edge_attr, W1[:, H2:H3], W2, b2.reshape(1, H))

    n_pad = ((n_nodes + NS * 8 - 1) // (NS * 8)) * (NS * 8)
    zeros = jnp.zeros((n_pad // NS, H), f32)
    agg0, agg1 = _make_scatter(n_pad, n_edges)(msg, dst, zeros)

    out = pl.pallas_call(
        _gru_body,
        grid=(n_nodes // bn,),
        in_specs=[pl.BlockSpec((bn, H), lambda i: (i, 0)),
                  pl.BlockSpec((bn, H), lambda i: (i, 0)),
                  pl.BlockSpec((bn, H), lambda i: (i, 0)),
                  pl.BlockSpec((H3, H), lambda i: (0, 0)),
                  pl.BlockSpec((H3, H), lambda i: (0, 0)),
                  pl.BlockSpec((1, H3), lambda i: (0, 0)),
                  pl.BlockSpec((1, H3), lambda i: (0, 0))],
        out_specs=pl.BlockSpec((bn, H), lambda i: (i, 0)),
        out_shape=jax.ShapeDtypeStruct((n_nodes, H), f32),
    )(agg0, agg1, x, Wih, Whh, bih.reshape(1, H3), bhh.reshape(1, H3))

    return out


# speedup vs baseline: 3.9538x; 1.0704x over previous
import functools

import jax
import jax.numpy as jnp
from jax import lax
from jax.experimental import pallas as pl
from jax.experimental.pallas import tpu as pltpu
from jax.experimental.pallas import tpu_sc as plsc

H = 128
H2 = 256
H3 = 384

NC = 2
NS = 16
NW = NC * NS

CH = 128

_DN = (((1,), (1,)), ((), ()))


def _node_pre_body(x_ref, w1_ref, b1_ref, s1_ref, s2_ref):
    xb = x_ref[...]
    w1 = w1_ref[...]
    s1_ref[...] = lax.dot_general(xb, w1[:, 0:H], _DN,
                                  preferred_element_type=jnp.float32) + b1_ref[...]
    s2_ref[...] = lax.dot_general(xb, w1[:, H:H2], _DN,
                                  preferred_element_type=jnp.float32)


def _edge_mlp_body(g1_ref, g2_ref, ea_ref, w1c_ref, w2_ref, b2_ref, out_ref):
    t = g1_ref[...] + g2_ref[...] + lax.dot_general(
        ea_ref[...], w1c_ref[...], _DN, preferred_element_type=jnp.float32)
    h = jnp.maximum(t, 0.0)
    out_ref[...] = lax.dot_general(
        h, w2_ref[...], _DN, preferred_element_type=jnp.float32) + b2_ref[...]


def _gru_body(a0_ref, a1_ref, x_ref, wih_ref, whh_ref, bih_ref, bhh_ref, out_ref):
    agg = a0_ref[...] + a1_ref[...]
    xb = x_ref[...]
    gi = lax.dot_general(agg, wih_ref[...], _DN,
                         preferred_element_type=jnp.float32) + bih_ref[...]
    gh = lax.dot_general(xb, whh_ref[...], _DN,
                         preferred_element_type=jnp.float32) + bhh_ref[...]
    r = jax.nn.sigmoid(gi[:, 0:H] + gh[:, 0:H])
    z = jax.nn.sigmoid(gi[:, H:H2] + gh[:, H:H2])
    n = jnp.tanh(gi[:, H2:H3] + r * gh[:, H2:H3])
    out_ref[...] = (1.0 - z) * n + z * xb


def _chunk_deal(wid, nchunks):
    base_ct = nchunks // NW
    rem = nchunks % NW
    extra = (wid < rem).astype(jnp.int32)
    start = base_ct * wid + jnp.minimum(wid, rem)
    return start, base_ct + extra


def _make_gather(n_nodes, n_edges):
    nchunks = n_edges // CH
    assert n_edges % CH == 0
    mesh = plsc.VectorSubcoreMesh(core_axis_name="c", subcore_axis_name="s")

    @functools.partial(
        pl.kernel, mesh=mesh,
        out_type=[jax.ShapeDtypeStruct((n_edges, H), jnp.float32),
                  jax.ShapeDtypeStruct((n_edges, H), jnp.float32)],
        scratch_types=[pltpu.VMEM((CH,), jnp.int32),
                       pltpu.VMEM((CH,), jnp.int32),
                       pltpu.VMEM((CH,), jnp.int32),
                       pltpu.VMEM((CH,), jnp.int32),
                       pltpu.VMEM((CH, H), jnp.float32),
                       pltpu.VMEM((CH, H), jnp.float32),
                       pltpu.VMEM((CH, H), jnp.float32),
                       pltpu.VMEM((CH, H), jnp.float32),
                       pltpu.SemaphoreType.DMA,
                       pltpu.SemaphoreType.DMA],
    )
    def gather_k(s1_hbm, s2_hbm, src_hbm, dst_hbm, o1_hbm, o2_hbm,
                 ia1, ia2, ib1, ib2, ba1, ba2, bb1, bb2, gsem, wsem):
        wid = lax.axis_index("s") * NC + lax.axis_index("c")
        start, my_ct = _chunk_deal(wid, nchunks)

        def do_chunk(base, idx1, idx2, buf1, buf2):
            pltpu.sync_copy(src_hbm.at[pl.ds(base, CH)], idx1)
            pltpu.sync_copy(dst_hbm.at[pl.ds(base, CH)], idx2)
            g1 = pltpu.async_copy(s1_hbm.at[idx1], buf1, gsem)
            g2 = pltpu.async_copy(s2_hbm.at[idx2], buf2, gsem)
            return g1, g2

        def put_chunk(base, buf1, buf2):
            w1 = pltpu.async_copy(buf1, o1_hbm.at[pl.ds(base, CH)], wsem)
            w2 = pltpu.async_copy(buf2, o2_hbm.at[pl.ds(base, CH)], wsem)
            return w1, w2

        def pair(p, carry):
            c0 = (start + 2 * p) * CH
            c1 = c0 + CH
            ga1, ga2 = do_chunk(c0, ia1, ia2, ba1, ba2)
            gb1, gb2 = do_chunk(c1, ib1, ib2, bb1, bb2)
            ga1.wait()
            ga2.wait()
            wa1, wa2 = put_chunk(c0, ba1, ba2)
            gb1.wait()
            gb2.wait()
            wb1, wb2 = put_chunk(c1, bb1, bb2)
            wa1.wait()
            wa2.wait()
            wb1.wait()
            wb2.wait()
            return carry

        lax.fori_loop(0, my_ct // 2, pair, 0)

        @pl.when(my_ct % 2 == 1)
        def _():
            c0 = (start + my_ct - 1) * CH
            g1, g2 = do_chunk(c0, ia1, ia2, ba1, ba2)
            g1.wait()
            g2.wait()
            w1, w2 = put_chunk(c0, ba1, ba2)
            w1.wait()
            w2.wait()

    return gather_k


def _make_scatter(n_pad, n_edges):
    nchunks = n_edges // CH
    rpt = n_pad // NS
    assert n_edges % CH == 0 and n_pad % NS == 0 and rpt % 8 == 0
    mesh = plsc.VectorSubcoreMesh(core_axis_name="c", subcore_axis_name="s")

    @functools.partial(
        pl.kernel, mesh=mesh,
        out_type=[jax.ShapeDtypeStruct((n_pad, H), jnp.float32),
                  jax.ShapeDtypeStruct((n_pad, H), jnp.float32)],
        scratch_types=[pltpu.VMEM((CH,), jnp.int32),
                       pltpu.VMEM((CH,), jnp.int32),
                       pltpu.VMEM((CH, H), jnp.float32),
                       pltpu.VMEM((CH, H), jnp.float32),
                       pltpu.VMEM_SHARED((n_pad, H), jnp.float32),
                       pltpu.SemaphoreType.DMA],
    )
    def scatter_k(msg_hbm, dst_hbm, zeros_hbm, agg0_hbm, agg1_hbm,
                  ia, ib, ma, mb, acc_spmem, ssem):
        cid = lax.axis_index("c")
        sid = lax.axis_index("s")
        wid = sid * NC + cid
        start, my_ct = _chunk_deal(wid, nchunks)

        pltpu.sync_copy(zeros_hbm, acc_spmem.at[pl.ds(sid * rpt, rpt)])
        plsc.subcore_barrier()

        def add_chunk(base, idx, mbuf):
            pltpu.sync_copy(dst_hbm.at[pl.ds(base, CH)], idx)
            pltpu.sync_copy(msg_hbm.at[pl.ds(base, CH)], mbuf)
            return pltpu.async_copy(mbuf, acc_spmem.at[idx], ssem, add=True)

        def pair(p, carry):
            c0 = (start + 2 * p) * CH
            aa = add_chunk(c0, ia, ma)
            ab = add_chunk(c0 + CH, ib, mb)
            aa.wait()
            ab.wait()
            return carry

        lax.fori_loop(0, my_ct // 2, pair, 0)

        @pl.when(my_ct % 2 == 1)
        def _():
            c0 = (start + my_ct - 1) * CH
            add_chunk(c0, ia, ma).wait()

        plsc.subcore_barrier()

        @pl.when(cid == 0)
        def _():
            pltpu.sync_copy(acc_spmem.at[pl.ds(sid * rpt, rpt)],
                            agg0_hbm.at[pl.ds(sid * rpt, rpt)])

        @pl.when(cid == 1)
        def _():
            pltpu.sync_copy(acc_spmem.at[pl.ds(sid * rpt, rpt)],
                            agg1_hbm.at[pl.ds(sid * rpt, rpt)])

    return scatter_k


def kernel(x, edge_index, edge_attr, W1, b1, W2, b2, Wih, Whh, bih, bhh):
    n_nodes, h = x.shape
    n_edges = edge_attr.shape[0]
    assert h == H

    src = edge_index[0].astype(jnp.int32)
    dst = edge_index[1].astype(jnp.int32)

    bn = 2000
    be = 2000
    assert n_nodes % bn == 0 and n_edges % be == 0

    f32 = jnp.float32

    s1t, s2t = pl.pallas_call(
        _node_pre_body,
        grid=(n_nodes // bn,),
        in_specs=[pl.BlockSpec((bn, H), lambda i: (i, 0)),
                  pl.BlockSpec((H, H3), lambda i: (0, 0)),
                  pl.BlockSpec((1, H), lambda i: (0, 0))],
        out_specs=[pl.BlockSpec((bn, H), lambda i: (i, 0)),
                   pl.BlockSpec((bn, H), lambda i: (i, 0))],
        out_shape=[jax.ShapeDtypeStruct((n_nodes, H), f32),
                   jax.ShapeDtypeStruct((n_nodes, H), f32)],
    )(x, W1, b1.reshape(1, H))

    g1, g2 = _make_gather(n_nodes, n_edges)(s1t, s2t, src, dst)

    msg = pl.pallas_call(
        _edge_mlp_body,
        grid=(n_edges // be,),
        in_specs=[pl.BlockSpec((be, H), lambda i: (i, 0)),
                  pl.BlockSpec((be, H), lambda i: (i, 0)),
                  pl.BlockSpec((be, H), lambda i: (i, 0)),
                  pl.BlockSpec((H, H), lambda i: (0, 0)),
                  pl.BlockSpec((H, H), lambda i: (0, 0)),
                  pl.BlockSpec((1, H), lambda i: (0, 0))],
        out_specs=pl.BlockSpec((be, H), lambda i: (i, 0)),
        out_shape=jax.ShapeDtypeStruct((n_edges, H), f32),
    )(g1, g2, edge_attr, W1[:, H2:H3], W2, b2.reshape(1, H))

    n_pad = ((n_nodes + NS * 8 - 1) // (NS * 8)) * (NS * 8)
    zeros = jnp.zeros((n_pad // NS, H), f32)
    agg0, agg1 = _make_scatter(n_pad, n_edges)(msg, dst, zeros)

    out = pl.pallas_call(
        _gru_body,
        grid=(n_nodes // bn,),
        in_specs=[pl.BlockSpec((bn, H), lambda i: (i, 0)),
                  pl.BlockSpec((bn, H), lambda i: (i, 0)),
                  pl.BlockSpec((bn, H), lambda i: (i, 0)),
                  pl.BlockSpec((H3, H), lambda i: (0, 0)),
                  pl.BlockSpec((H3, H), lambda i: (0, 0)),
                  pl.BlockSpec((1, H3), lambda i: (0, 0)),
                  pl.BlockSpec((1, H3), lambda i: (0, 0))],
        out_specs=pl.BlockSpec((bn, H), lambda i: (i, 0)),
        out_shape=jax.ShapeDtypeStruct((n_nodes, H), f32),
    )(agg0, agg1, x, Wih, Whh, bih.reshape(1, H3), bhh.reshape(1, H3))

    return out


# speedup vs baseline: 4.5052x; 1.1395x over previous
import functools

import jax
import jax.numpy as jnp
from jax import lax
from jax.experimental import pallas as pl
from jax.experimental.pallas import tpu as pltpu
from jax.experimental.pallas import tpu_sc as plsc

H = 128
H2 = 256
H3 = 384

NC = 2
NS = 16
NW = NC * NS

CH = 128

_DN = (((1,), (1,)), ((), ()))


def _node_pre_body(x_ref, w1_ref, b1_ref, s1_ref, s2_ref):
    xb = x_ref[...]
    w1 = w1_ref[...]
    s1_ref[...] = lax.dot_general(xb, w1[:, 0:H], _DN,
                                  preferred_element_type=jnp.float32) + b1_ref[...]
    s2_ref[...] = lax.dot_general(xb, w1[:, H:H2], _DN,
                                  preferred_element_type=jnp.float32)


def _edge_mlp_body(g1_ref, g2_ref, ea_ref, w1c_ref, w2_ref, b2_ref, out_ref):
    t = g1_ref[...] + g2_ref[...] + lax.dot_general(
        ea_ref[...], w1c_ref[...], _DN, preferred_element_type=jnp.float32)
    h = jnp.maximum(t, 0.0)
    out_ref[...] = lax.dot_general(
        h, w2_ref[...], _DN, preferred_element_type=jnp.float32) + b2_ref[...]


def _gru_body(a0_ref, a1_ref, a2_ref, a3_ref, x_ref, wih_ref, whh_ref,
              bih_ref, bhh_ref, out_ref):
    agg = (a0_ref[...] + a1_ref[...]) + (a2_ref[...] + a3_ref[...])
    xb = x_ref[...]
    gi = lax.dot_general(agg, wih_ref[...], _DN,
                         preferred_element_type=jnp.float32) + bih_ref[...]
    gh = lax.dot_general(xb, whh_ref[...], _DN,
                         preferred_element_type=jnp.float32) + bhh_ref[...]
    r = jax.nn.sigmoid(gi[:, 0:H] + gh[:, 0:H])
    z = jax.nn.sigmoid(gi[:, H:H2] + gh[:, H:H2])
    n = jnp.tanh(gi[:, H2:H3] + r * gh[:, H2:H3])
    out_ref[...] = (1.0 - z) * n + z * xb


def _chunk_deal(wid, nchunks):
    base_ct = nchunks // NW
    rem = nchunks % NW
    extra = (wid < rem).astype(jnp.int32)
    start = base_ct * wid + jnp.minimum(wid, rem)
    return start, base_ct + extra


def _make_gather(n_nodes, n_edges, chunk_off=0):
    nchunks = n_edges // CH
    assert n_edges % CH == 0
    mesh = plsc.VectorSubcoreMesh(core_axis_name="c", subcore_axis_name="s")

    @functools.partial(
        pl.kernel, mesh=mesh,
        out_type=[jax.ShapeDtypeStruct((n_edges, H), jnp.float32),
                  jax.ShapeDtypeStruct((n_edges, H), jnp.float32)],
        scratch_types=[pltpu.VMEM((CH,), jnp.int32),
                       pltpu.VMEM((CH,), jnp.int32),
                       pltpu.VMEM((CH,), jnp.int32),
                       pltpu.VMEM((CH,), jnp.int32),
                       pltpu.VMEM((CH, H), jnp.float32),
                       pltpu.VMEM((CH, H), jnp.float32),
                       pltpu.VMEM((CH, H), jnp.float32),
                       pltpu.VMEM((CH, H), jnp.float32),
                       pltpu.SemaphoreType.DMA,
                       pltpu.SemaphoreType.DMA],
    )
    def gather_k(s1_hbm, s2_hbm, src_hbm, dst_hbm, o1_hbm, o2_hbm,
                 ia1, ia2, ib1, ib2, ba1, ba2, bb1, bb2, gsem, wsem):
        wid = lax.axis_index("s") * NC + lax.axis_index("c")
        start, my_ct = _chunk_deal(wid, nchunks)

        def do_chunk(base, idx1, idx2, buf1, buf2):
            gbase = base + chunk_off * CH
            pltpu.sync_copy(src_hbm.at[pl.ds(gbase, CH)], idx1)
            pltpu.sync_copy(dst_hbm.at[pl.ds(gbase, CH)], idx2)
            g1 = pltpu.async_copy(s1_hbm.at[idx1], buf1, gsem)
            g2 = pltpu.async_copy(s2_hbm.at[idx2], buf2, gsem)
            return g1, g2

        def put_chunk(base, buf1, buf2):
            w1 = pltpu.async_copy(buf1, o1_hbm.at[pl.ds(base, CH)], wsem)
            w2 = pltpu.async_copy(buf2, o2_hbm.at[pl.ds(base, CH)], wsem)
            return w1, w2

        def pair(p, carry):
            c0 = (start + 2 * p) * CH
            c1 = c0 + CH
            ga1, ga2 = do_chunk(c0, ia1, ia2, ba1, ba2)
            gb1, gb2 = do_chunk(c1, ib1, ib2, bb1, bb2)
            ga1.wait()
            ga2.wait()
            wa1, wa2 = put_chunk(c0, ba1, ba2)
            gb1.wait()
            gb2.wait()
            wb1, wb2 = put_chunk(c1, bb1, bb2)
            wa1.wait()
            wa2.wait()
            wb1.wait()
            wb2.wait()
            return carry

        lax.fori_loop(0, my_ct // 2, pair, 0)

        @pl.when(my_ct % 2 == 1)
        def _():
            c0 = (start + my_ct - 1) * CH
            g1, g2 = do_chunk(c0, ia1, ia2, ba1, ba2)
            g1.wait()
            g2.wait()
            w1, w2 = put_chunk(c0, ba1, ba2)
            w1.wait()
            w2.wait()

    return gather_k


def _make_scatter(n_pad, n_edges, chunk_off=0):
    nchunks = n_edges // CH
    rpt = n_pad // NS
    assert n_edges % CH == 0 and n_pad % NS == 0 and rpt % 8 == 0
    mesh = plsc.VectorSubcoreMesh(core_axis_name="c", subcore_axis_name="s")

    @functools.partial(
        pl.kernel, mesh=mesh,
        out_type=[jax.ShapeDtypeStruct((n_pad, H), jnp.float32),
                  jax.ShapeDtypeStruct((n_pad, H), jnp.float32)],
        scratch_types=[pltpu.VMEM((CH,), jnp.int32),
                       pltpu.VMEM((CH,), jnp.int32),
                       pltpu.VMEM((CH, H), jnp.float32),
                       pltpu.VMEM((CH, H), jnp.float32),
                       pltpu.VMEM_SHARED((n_pad, H), jnp.float32),
                       pltpu.SemaphoreType.DMA],
    )
    def scatter_k(msg_hbm, dst_hbm, zeros_hbm, agg0_hbm, agg1_hbm,
                  ia, ib, ma, mb, acc_spmem, ssem):
        cid = lax.axis_index("c")
        sid = lax.axis_index("s")
        wid = sid * NC + cid
        start, my_ct = _chunk_deal(wid, nchunks)

        pltpu.sync_copy(zeros_hbm, acc_spmem.at[pl.ds(sid * rpt, rpt)])
        plsc.subcore_barrier()

        def add_chunk(base, idx, mbuf):
            pltpu.sync_copy(dst_hbm.at[pl.ds(base + chunk_off * CH, CH)], idx)
            pltpu.sync_copy(msg_hbm.at[pl.ds(base, CH)], mbuf)
            return pltpu.async_copy(mbuf, acc_spmem.at[idx], ssem, add=True)

        def pair(p, carry):
            c0 = (start + 2 * p) * CH
            aa = add_chunk(c0, ia, ma)
            ab = add_chunk(c0 + CH, ib, mb)
            aa.wait()
            ab.wait()
            return carry

        lax.fori_loop(0, my_ct // 2, pair, 0)

        @pl.when(my_ct % 2 == 1)
        def _():
            c0 = (start + my_ct - 1) * CH
            add_chunk(c0, ia, ma).wait()

        plsc.subcore_barrier()

        @pl.when(cid == 0)
        def _():
            pltpu.sync_copy(acc_spmem.at[pl.ds(sid * rpt, rpt)],
                            agg0_hbm.at[pl.ds(sid * rpt, rpt)])

        @pl.when(cid == 1)
        def _():
            pltpu.sync_copy(acc_spmem.at[pl.ds(sid * rpt, rpt)],
                            agg1_hbm.at[pl.ds(sid * rpt, rpt)])

    return scatter_k


def kernel(x, edge_index, edge_attr, W1, b1, W2, b2, Wih, Whh, bih, bhh):
    n_nodes, h = x.shape
    n_edges = edge_attr.shape[0]
    assert h == H

    src = edge_index[0].astype(jnp.int32)
    dst = edge_index[1].astype(jnp.int32)

    bn = 2000
    be = 2000
    assert n_nodes % bn == 0 and n_edges % be == 0

    f32 = jnp.float32

    s1t, s2t = pl.pallas_call(
        _node_pre_body,
        grid=(n_nodes // bn,),
        in_specs=[pl.BlockSpec((bn, H), lambda i: (i, 0)),
                  pl.BlockSpec((H, H3), lambda i: (0, 0)),
                  pl.BlockSpec((1, H), lambda i: (0, 0))],
        out_specs=[pl.BlockSpec((bn, H), lambda i: (i, 0)),
                   pl.BlockSpec((bn, H), lambda i: (i, 0))],
        out_shape=[jax.ShapeDtypeStruct((n_nodes, H), f32),
                   jax.ShapeDtypeStruct((n_nodes, H), f32)],
    )(x, W1, b1.reshape(1, H))

    n_pad = ((n_nodes + NS * 8 - 1) // (NS * 8)) * (NS * 8)
    zeros = jnp.zeros((n_pad // NS, H), f32)
    n_sh = n_edges // 2
    assert n_sh % CH == 0 and n_sh % be == 0
    sh_blocks = n_sh // be

    aggs = []
    for s in range(2):
        off = s * (n_sh // CH)
        g1, g2 = _make_gather(n_nodes, n_sh, off)(s1t, s2t, src, dst)
        msg = pl.pallas_call(
            _edge_mlp_body,
            grid=(sh_blocks,),
            in_specs=[pl.BlockSpec((be, H), lambda i: (i, 0)),
                      pl.BlockSpec((be, H), lambda i: (i, 0)),
                      pl.BlockSpec((be, H),
                                   lambda i, o=s * sh_blocks: (i + o, 0)),
                      pl.BlockSpec((H, H), lambda i: (0, 0)),
                      pl.BlockSpec((H, H), lambda i: (0, 0)),
                      pl.BlockSpec((1, H), lambda i: (0, 0))],
            out_specs=pl.BlockSpec((be, H), lambda i: (i, 0)),
            out_shape=jax.ShapeDtypeStruct((n_sh, H), f32),
        )(g1, g2, edge_attr, W1[:, H2:H3], W2, b2.reshape(1, H))
        a0, a1 = _make_scatter(n_pad, n_sh, off)(msg, dst, zeros)
        aggs += [a0, a1]

    out = pl.pallas_call(
        _gru_body,
        grid=(n_nodes // bn,),
        in_specs=[pl.BlockSpec((bn, H), lambda i: (i, 0)),
                  pl.BlockSpec((bn, H), lambda i: (i, 0)),
                  pl.BlockSpec((bn, H), lambda i: (i, 0)),
                  pl.BlockSpec((bn, H), lambda i: (i, 0)),
                  pl.BlockSpec((bn, H), lambda i: (i, 0)),
                  pl.BlockSpec((H3, H), lambda i: (0, 0)),
                  pl.BlockSpec((H3, H), lambda i: (0, 0)),
                  pl.BlockSpec((1, H3), lambda i: (0, 0)),
                  pl.BlockSpec((1, H3), lambda i: (0, 0))],
        out_specs=pl.BlockSpec((bn, H), lambda i: (i, 0)),
        out_shape=jax.ShapeDtypeStruct((n_nodes, H), f32),
    )(*aggs, x, Wih, Whh, bih.reshape(1, H3), bhh.reshape(1, H3))

    return out
